# baseline (speedup 1.0000x reference)
import functools

import jax
import jax.numpy as jnp
from jax.experimental import pallas as pl
from jax.experimental.pallas import tpu as pltpu

_PREC = jax.lax.Precision.DEFAULT


def _router_kernel(nf, k_top, x_ref, tef_ref, tet_ref, w_in_ref, b_in_ref,
                   w_mid_ref, b_mid_ref, w_r_ref, b_r_ref, ap_ref,
                   probs_ref, mask_ref, ent_ref, acc_ref):
    i = pl.program_id(0)
    j = pl.program_id(1)
    h_dim = x_ref.shape[1]

    w_blk = w_in_ref[...]
    h1 = jnp.dot(x_ref[...], w_blk[:h_dim, :],
                 preferred_element_type=jnp.float32, precision=_PREC)
    h1 = h1 + jnp.dot(tef_ref[...], w_blk[h_dim:, :],
                      preferred_element_type=jnp.float32, precision=_PREC)
    h1 = jnp.maximum(h1 + b_in_ref[...], 0.0)
    contrib = jnp.dot(h1, w_mid_ref[...],
                      preferred_element_type=jnp.float32, precision=_PREC)

    @pl.when(j == 0)
    def _():
        acc_ref[...] = contrib

    @pl.when(j > 0)
    def _():
        acc_ref[...] = acc_ref[...] + contrib

    @pl.when(j == nf - 1)
    def _():
        h2 = jnp.maximum(acc_ref[...] + b_mid_ref[...], 0.0)
        logits = jnp.dot(h2, w_r_ref[...],
                         preferred_element_type=jnp.float32,
                         precision=_PREC) + b_r_ref[...]
        ep = jax.nn.softmax(logits, axis=-1)

        t_count = tet_ref.shape[0]
        ap_w = ap_ref[...]
        att = None
        for t in range(t_count):
            s = jnp.dot(tet_ref[t], ap_w,
                        preferred_element_type=jnp.float32, precision=_PREC)
            sm = jax.nn.softmax(s, axis=-1)
            att = sm if att is None else att + sm
        att = att * (1.0 / t_count)

        p = ep * att
        e_dim = p.shape[-1]
        idx = jax.lax.broadcasted_iota(jnp.int32, p.shape, 1)
        vals = p
        msk = jnp.zeros_like(p)
        for _ in range(k_top):
            m = jnp.max(vals, axis=-1, keepdims=True)
            is_max = vals == m
            sel_idx = jnp.min(jnp.where(is_max, idx, e_dim), axis=-1,
                              keepdims=True)
            sel = idx == sel_idx
            msk = jnp.where(sel, 1.0, msk)
            vals = jnp.where(sel, -jnp.inf, vals)

        pm = p * msk
        probs_ref[...] = pm
        mask_ref[...] = msk
        ent_part = jnp.sum(pm * jnp.log(pm + 1e-8))[None, None]

        @pl.when(i == 0)
        def _():
            ent_ref[...] = ent_part

        @pl.when(i > 0)
        def _():
            ent_ref[...] = ent_ref[...] + ent_part


def kernel(x, task_embeddings, attribute_proj, W_in, b_in, W_mid, b_mid,
           W_r, b_r):
    B, S, H = x.shape
    T, TD = task_embeddings.shape[2], task_embeddings.shape[3]
    N = B * S
    FF = W_in.shape[1]
    E = W_r.shape[1]
    K = 8

    x2 = x.reshape(N, H)
    tef = task_embeddings.reshape(N, T * TD)
    tet = jnp.transpose(task_embeddings.reshape(N, T, TD), (1, 0, 2))
    b_in2 = b_in.reshape(1, FF)
    b_mid2 = b_mid.reshape(1, H)
    b_r2 = b_r.reshape(1, E)

    BT = min(1024, N)
    BF = min(256, FF)
    nt, nf = N // BT, FF // BF

    probs, msk, ent = pl.pallas_call(
        functools.partial(_router_kernel, nf, K),
        grid=(nt, nf),
        in_specs=[
            pl.BlockSpec((BT, H), lambda i, j: (i, 0)),
            pl.BlockSpec((BT, T * TD), lambda i, j: (i, 0)),
            pl.BlockSpec((T, BT, TD), lambda i, j: (0, i, 0)),
            pl.BlockSpec((H + T * TD, BF), lambda i, j: (0, j)),
            pl.BlockSpec((1, BF), lambda i, j: (0, j)),
            pl.BlockSpec((BF, H), lambda i, j: (j, 0)),
            pl.BlockSpec((1, H), lambda i, j: (0, 0)),
            pl.BlockSpec((H, E), lambda i, j: (0, 0)),
            pl.BlockSpec((1, E), lambda i, j: (0, 0)),
            pl.BlockSpec((TD, E), lambda i, j: (0, 0)),
        ],
        out_specs=[
            pl.BlockSpec((BT, E), lambda i, j: (i, 0)),
            pl.BlockSpec((BT, E), lambda i, j: (i, 0)),
            pl.BlockSpec((1, 1), lambda i, j: (0, 0)),
        ],
        out_shape=[
            jax.ShapeDtypeStruct((N, E), jnp.float32),
            jax.ShapeDtypeStruct((N, E), jnp.float32),
            jax.ShapeDtypeStruct((1, 1), jnp.float32),
        ],
        scratch_shapes=[pltpu.VMEM((BT, H), jnp.float32)],
        compiler_params=pltpu.CompilerParams(
            dimension_semantics=("arbitrary", "arbitrary"),
        ),
    )(x2, tef, tet, W_in, b_in2, W_mid, b_mid2, W_r, b_r2, attribute_proj)

    expert_probs = probs.reshape(B, S, E)
    mask = msk.reshape(B, S, E)
    entropy_loss = -(ent[0, 0] / N)
    return expert_probs, entropy_loss, mask


# speedup vs baseline: 1.6655x; 1.6655x over previous
import functools

import jax
import jax.numpy as jnp
from jax.experimental import pallas as pl
from jax.experimental.pallas import tpu as pltpu

_PREC = jax.lax.Precision.DEFAULT


def _router_kernel(nf, k_top, x_ref, tef_ref, tet_ref, w_in_ref, b_in_ref,
                   w_mid_ref, b_mid_ref, w_r_ref, b_r_ref, ap_ref,
                   probs_ref, mask_ref, ent_ref, acc_ref):
    i = pl.program_id(0)
    j = pl.program_id(1)
    h_dim = x_ref.shape[1]

    w_blk = w_in_ref[...]
    h1 = jnp.dot(x_ref[...], w_blk[:h_dim, :],
                 preferred_element_type=jnp.float32, precision=_PREC)
    h1 = h1 + jnp.dot(tef_ref[...], w_blk[h_dim:, :],
                      preferred_element_type=jnp.float32, precision=_PREC)
    h1 = jnp.maximum(h1 + b_in_ref[...], 0.0).astype(jnp.bfloat16)
    contrib = jnp.dot(h1, w_mid_ref[...],
                      preferred_element_type=jnp.float32, precision=_PREC)

    @pl.when(j == 0)
    def _():
        acc_ref[...] = contrib

    @pl.when(j > 0)
    def _():
        acc_ref[...] = acc_ref[...] + contrib

    @pl.when(j == nf - 1)
    def _():
        h2 = jnp.maximum(acc_ref[...] + b_mid_ref[...], 0.0)
        logits = jnp.dot(h2, w_r_ref[...],
                         preferred_element_type=jnp.float32,
                         precision=_PREC) + b_r_ref[...]
        ep = jax.nn.softmax(logits, axis=-1)

        t_count = tet_ref.shape[0]
        ap_w = ap_ref[...]
        att = None
        for t in range(t_count):
            s = jnp.dot(tet_ref[t], ap_w,
                        preferred_element_type=jnp.float32, precision=_PREC)
            sm = jax.nn.softmax(s, axis=-1)
            att = sm if att is None else att + sm
        att = att * (1.0 / t_count)

        p = ep * att
        e_dim = p.shape[-1]
        idx = jax.lax.broadcasted_iota(jnp.int32, p.shape, 1)
        vals = p
        msk = jnp.zeros_like(p)
        for _ in range(k_top):
            m = jnp.max(vals, axis=-1, keepdims=True)
            is_max = vals == m
            sel_idx = jnp.min(jnp.where(is_max, idx, e_dim), axis=-1,
                              keepdims=True)
            sel = idx == sel_idx
            msk = jnp.where(sel, 1.0, msk)
            vals = jnp.where(sel, -jnp.inf, vals)

        pm = p * msk
        probs_ref[...] = pm
        mask_ref[...] = msk
        ent_part = jnp.sum(pm * jnp.log(pm + 1e-8))[None, None]

        @pl.when(i == 0)
        def _():
            ent_ref[...] = ent_part

        @pl.when(i > 0)
        def _():
            ent_ref[...] = ent_ref[...] + ent_part


def kernel(x, task_embeddings, attribute_proj, W_in, b_in, W_mid, b_mid,
           W_r, b_r):
    B, S, H = x.shape
    T, TD = task_embeddings.shape[2], task_embeddings.shape[3]
    N = B * S
    FF = W_in.shape[1]
    E = W_r.shape[1]
    K = 8

    x2 = x.reshape(N, H).astype(jnp.bfloat16)
    tef = task_embeddings.reshape(N, T * TD).astype(jnp.bfloat16)
    tet = jnp.transpose(task_embeddings.reshape(N, T, TD), (1, 0, 2))
    w_in_b = W_in.astype(jnp.bfloat16)
    w_mid_b = W_mid.astype(jnp.bfloat16)
    b_in2 = b_in.reshape(1, FF)
    b_mid2 = b_mid.reshape(1, H)
    b_r2 = b_r.reshape(1, E)

    BT = min(1024, N)
    BF = min(1024, FF)
    nt, nf = N // BT, FF // BF

    probs, msk, ent = pl.pallas_call(
        functools.partial(_router_kernel, nf, K),
        grid=(nt, nf),
        in_specs=[
            pl.BlockSpec((BT, H), lambda i, j: (i, 0)),
            pl.BlockSpec((BT, T * TD), lambda i, j: (i, 0)),
            pl.BlockSpec((T, BT, TD), lambda i, j: (0, i, 0)),
            pl.BlockSpec((H + T * TD, BF), lambda i, j: (0, j)),
            pl.BlockSpec((1, BF), lambda i, j: (0, j)),
            pl.BlockSpec((BF, H), lambda i, j: (j, 0)),
            pl.BlockSpec((1, H), lambda i, j: (0, 0)),
            pl.BlockSpec((H, E), lambda i, j: (0, 0)),
            pl.BlockSpec((1, E), lambda i, j: (0, 0)),
            pl.BlockSpec((TD, E), lambda i, j: (0, 0)),
        ],
        out_specs=[
            pl.BlockSpec((BT, E), lambda i, j: (i, 0)),
            pl.BlockSpec((BT, E), lambda i, j: (i, 0)),
            pl.BlockSpec((1, 1), lambda i, j: (0, 0)),
        ],
        out_shape=[
            jax.ShapeDtypeStruct((N, E), jnp.float32),
            jax.ShapeDtypeStruct((N, E), jnp.float32),
            jax.ShapeDtypeStruct((1, 1), jnp.float32),
        ],
        scratch_shapes=[pltpu.VMEM((BT, H), jnp.float32)],
        compiler_params=pltpu.CompilerParams(
            dimension_semantics=("arbitrary", "arbitrary"),
        ),
    )(x2, tef, tet, w_in_b, b_in2, w_mid_b, b_mid2, W_r, b_r2,
      attribute_proj)

    expert_probs = probs.reshape(B, S, E)
    mask = msk.reshape(B, S, E)
    entropy_loss = -(ent[0, 0] / N)
    return expert_probs, entropy_loss, mask


# speedup vs baseline: 1.6870x; 1.0129x over previous
import functools

import jax
import jax.numpy as jnp
from jax.experimental import pallas as pl
from jax.experimental.pallas import tpu as pltpu

_PREC = jax.lax.Precision.DEFAULT


def _router_kernel(nf, k_top, t_count, x_ref, tef_ref, w_in_ref, b_in_ref,
                   w_mid_ref, b_mid_ref, w_r_ref, b_r_ref, ap_ref,
                   probs_ref, mask_ref, ent_ref, acc_ref):
    i = pl.program_id(0)
    j = pl.program_id(1)
    h_dim = x_ref.shape[1]

    w_blk = w_in_ref[...]
    h1 = jnp.dot(x_ref[...], w_blk[:h_dim, :],
                 preferred_element_type=jnp.float32, precision=_PREC)
    h1 = h1 + jnp.dot(tef_ref[...], w_blk[h_dim:, :],
                      preferred_element_type=jnp.float32, precision=_PREC)
    h1 = jnp.maximum(h1 + b_in_ref[...], 0.0).astype(jnp.bfloat16)
    contrib = jnp.dot(h1, w_mid_ref[...],
                      preferred_element_type=jnp.float32, precision=_PREC)

    @pl.when(j == 0)
    def _():
        acc_ref[...] = contrib

    @pl.when(j > 0)
    def _():
        acc_ref[...] = acc_ref[...] + contrib

    @pl.when(j == nf - 1)
    def _():
        h2 = jnp.maximum(acc_ref[...] + b_mid_ref[...], 0.0)
        logits = jnp.dot(h2, w_r_ref[...],
                         preferred_element_type=jnp.float32,
                         precision=_PREC) + b_r_ref[...]
        ep = jax.nn.softmax(logits, axis=-1)

        td = ap_ref.shape[0]
        ap_w = ap_ref[...].astype(jnp.bfloat16)
        att = None
        for t in range(t_count):
            s = jnp.dot(tef_ref[:, t * td:(t + 1) * td], ap_w,
                        preferred_element_type=jnp.float32, precision=_PREC)
            sm = jax.nn.softmax(s, axis=-1)
            att = sm if att is None else att + sm
        att = att * (1.0 / t_count)

        p = ep * att
        e_dim = p.shape[-1]
        idx = jax.lax.broadcasted_iota(jnp.int32, p.shape, 1)
        vals = p
        msk = jnp.zeros_like(p)
        for _ in range(k_top):
            m = jnp.max(vals, axis=-1, keepdims=True)
            is_max = vals == m
            sel_idx = jnp.min(jnp.where(is_max, idx, e_dim), axis=-1,
                              keepdims=True)
            sel = idx == sel_idx
            msk = jnp.where(sel, 1.0, msk)
            vals = jnp.where(sel, -jnp.inf, vals)

        pm = p * msk
        probs_ref[...] = pm
        mask_ref[...] = msk
        ent_part = jnp.sum(pm * jnp.log(pm + 1e-8))[None, None]

        @pl.when(i == 0)
        def _():
            ent_ref[...] = ent_part

        @pl.when(i > 0)
        def _():
            ent_ref[...] = ent_ref[...] + ent_part


def kernel(x, task_embeddings, attribute_proj, W_in, b_in, W_mid, b_mid,
           W_r, b_r):
    B, S, H = x.shape
    T, TD = task_embeddings.shape[2], task_embeddings.shape[3]
    N = B * S
    FF = W_in.shape[1]
    E = W_r.shape[1]
    K = 8

    x2 = x.reshape(N, H).astype(jnp.bfloat16)
    tef = task_embeddings.reshape(N, T * TD).astype(jnp.bfloat16)
    w_in_b = W_in.astype(jnp.bfloat16)
    w_mid_b = W_mid.astype(jnp.bfloat16)
    b_in2 = b_in.reshape(1, FF)
    b_mid2 = b_mid.reshape(1, H)
    b_r2 = b_r.reshape(1, E)

    BT = min(1024, N)
    BF = min(1024, FF)
    nt, nf = N // BT, FF // BF

    probs, msk, ent = pl.pallas_call(
        functools.partial(_router_kernel, nf, K, T),
        grid=(nt, nf),
        in_specs=[
            pl.BlockSpec((BT, H), lambda i, j: (i, 0)),
            pl.BlockSpec((BT, T * TD), lambda i, j: (i, 0)),
            pl.BlockSpec((H + T * TD, BF), lambda i, j: (0, j)),
            pl.BlockSpec((1, BF), lambda i, j: (0, j)),
            pl.BlockSpec((BF, H), lambda i, j: (j, 0)),
            pl.BlockSpec((1, H), lambda i, j: (0, 0)),
            pl.BlockSpec((H, E), lambda i, j: (0, 0)),
            pl.BlockSpec((1, E), lambda i, j: (0, 0)),
            pl.BlockSpec((TD, E), lambda i, j: (0, 0)),
        ],
        out_specs=[
            pl.BlockSpec((BT, E), lambda i, j: (i, 0)),
            pl.BlockSpec((BT, E), lambda i, j: (i, 0)),
            pl.BlockSpec((1, 1), lambda i, j: (0, 0)),
        ],
        out_shape=[
            jax.ShapeDtypeStruct((N, E), jnp.float32),
            jax.ShapeDtypeStruct((N, E), jnp.float32),
            jax.ShapeDtypeStruct((1, 1), jnp.float32),
        ],
        scratch_shapes=[pltpu.VMEM((BT, H), jnp.float32)],
        compiler_params=pltpu.CompilerParams(
            dimension_semantics=("arbitrary", "arbitrary"),
        ),
    )(x2, tef, w_in_b, b_in2, w_mid_b, b_mid2, W_r, b_r2,
      attribute_proj)

    expert_probs = probs.reshape(B, S, E)
    mask = msk.reshape(B, S, E)
    entropy_loss = -(ent[0, 0] / N)
    return expert_probs, entropy_loss, mask
